# Initial kernel scaffold; baseline (speedup 1.0000x reference)
import jax
import jax.numpy as jnp
from jax.experimental import pallas as pl


def kernel(z, e):
    raise NotImplementedError("write your pallas kernel here")


# speedup vs baseline: 3.3897x; 3.3897x over previous
import functools

import jax
import jax.numpy as jnp
from jax import lax
from jax.experimental import pallas as pl
from jax.experimental.pallas import tpu as pltpu
from jax.experimental.pallas import tpu_sc as plsc

N_NODES = 10000
D = 128
E = 320000
LANES = 16
CHUNKS = D // LANES

_info = plsc.get_sparse_core_info()
NC, NS = _info.num_cores, _info.num_subcores
NW = NC * NS
EDGES_PER_W = E // NW
B = 80
NBLOCKS = EDGES_PER_W // B
GROUPS = B // LANES

_mesh = plsc.VectorSubcoreMesh(core_axis_name="c", subcore_axis_name="s")


@functools.partial(
    pl.kernel,
    out_type=jax.ShapeDtypeStruct((E,), jnp.float32),
    mesh=_mesh,
    compiler_params=pltpu.CompilerParams(needs_layout_passes=False),
    scratch_types=[
        pltpu.VMEM((B,), jnp.int32),
        pltpu.VMEM((B,), jnp.int32),
        pltpu.VMEM((B, D), jnp.float32),
        pltpu.VMEM((B, D), jnp.float32),
        pltpu.VMEM((LANES * LANES,), jnp.float32),
        pltpu.VMEM((B,), jnp.float32),
        pltpu.SemaphoreType.DMA,
        pltpu.SemaphoreType.DMA,
    ],
)
def _sc_kernel(z_hbm, e0_hbm, e1_hbm, out_hbm,
               idx0_v, idx1_v, rows0_v, rows1_v, pv, out_v, sem0, sem1):
    wid = lax.axis_index("s") * NC + lax.axis_index("c")
    iot = lax.iota(jnp.int32, LANES)

    def group_body(g, _):
        for u in range(LANES):
            i = g * LANES + u
            acc = None
            for c in range(CHUNKS):
                a = jnp.maximum(rows0_v[i, pl.ds(c * LANES, LANES)], 0.0)
                b = jnp.maximum(rows1_v[i, pl.ds(c * LANES, LANES)], 0.0)
                p = a * b
                acc = p if acc is None else acc + p
            pv[pl.ds(u * LANES, LANES)] = acc
        total = None
        stride_idx = iot * LANES
        for j in range(LANES):
            col = plsc.load_gather(pv, [stride_idx + j])
            total = col if total is None else total + col
        out_v[pl.ds(g * LANES, LANES)] = 1.0 / (1.0 + jnp.exp(-total))
        return 0

    def block_body(t, _):
        base = pl.multiple_of(wid * EDGES_PER_W + t * B, 16)
        pltpu.sync_copy(e0_hbm.at[pl.ds(base, B)], idx0_v)
        pltpu.sync_copy(e1_hbm.at[pl.ds(base, B)], idx1_v)
        cp0 = pltpu.async_copy(z_hbm.at[idx0_v], rows0_v, sem0)
        cp1 = pltpu.async_copy(z_hbm.at[idx1_v], rows1_v, sem1)
        cp0.wait()
        cp1.wait()
        lax.fori_loop(0, GROUPS, group_body, 0)
        pltpu.sync_copy(out_v, out_hbm.at[pl.ds(base, B)])
        return 0

    lax.fori_loop(0, NBLOCKS, block_body, 0)


def kernel(z, e):
    e0 = e[0]
    e1 = e[1]
    return _sc_kernel(z, e0, e1)


# speedup vs baseline: 7.5429x; 2.2253x over previous
import functools

import jax
import jax.numpy as jnp
from jax import lax
from jax.experimental import pallas as pl
from jax.experimental.pallas import tpu as pltpu
from jax.experimental.pallas import tpu_sc as plsc

N_NODES = 10000
D = 128
E = 320000
LANES = 16
CHUNKS = D // LANES

_info = plsc.get_sparse_core_info()
NC, NS = _info.num_cores, _info.num_subcores
NW = NC * NS
EDGES_PER_W = E // NW
B = 80
NBLOCKS = EDGES_PER_W // B
GROUPS = B // LANES

_mesh = plsc.VectorSubcoreMesh(core_axis_name="c", subcore_axis_name="s")


@functools.partial(
    pl.kernel,
    out_type=jax.ShapeDtypeStruct((E,), jnp.float32),
    mesh=_mesh,
    compiler_params=pltpu.CompilerParams(needs_layout_passes=False),
    scratch_types=[
        pltpu.VMEM((EDGES_PER_W,), jnp.int32),
        pltpu.VMEM((EDGES_PER_W,), jnp.int32),
        pltpu.VMEM((2, B, D), jnp.float32),
        pltpu.VMEM((2, B, D), jnp.float32),
        pltpu.VMEM((LANES * LANES,), jnp.float32),
        pltpu.VMEM((EDGES_PER_W,), jnp.float32),
        pltpu.SemaphoreType.DMA,
        pltpu.SemaphoreType.DMA,
        pltpu.SemaphoreType.DMA,
        pltpu.SemaphoreType.DMA,
    ],
)
def _sc_kernel(z_hbm, e0_hbm, e1_hbm, out_hbm,
               idxa, idxb, rows0, rows1, pv, outb, s0a, s0b, s1a, s1b):
    wid = lax.axis_index("s") * NC + lax.axis_index("c")
    ebase = pl.multiple_of(wid * EDGES_PER_W, 16)
    iot = lax.iota(jnp.int32, LANES)

    def issue(t, buf, sa, sb):
        off = pl.multiple_of(t * B, 16)
        pltpu.async_copy(z_hbm.at[idxa.at[pl.ds(off, B)]], rows0.at[buf], sa)
        pltpu.async_copy(z_hbm.at[idxb.at[pl.ds(off, B)]], rows1.at[buf], sb)

    def wait(t, buf, sa, sb):
        off = pl.multiple_of(t * B, 16)
        pltpu.make_async_copy(
            z_hbm.at[idxa.at[pl.ds(off, B)]], rows0.at[buf], sa).wait()
        pltpu.make_async_copy(
            z_hbm.at[idxb.at[pl.ds(off, B)]], rows1.at[buf], sb).wait()

    def compute(t, buf):
        r0 = rows0.at[buf]
        r1 = rows1.at[buf]

        def group_body(g, _):
            for u in range(LANES):
                i = g * LANES + u
                acc = None
                for c in range(CHUNKS):
                    a = jnp.maximum(r0[i, pl.ds(c * LANES, LANES)], 0.0)
                    b = jnp.maximum(r1[i, pl.ds(c * LANES, LANES)], 0.0)
                    p = a * b
                    acc = p if acc is None else acc + p
                pv[pl.ds(u * LANES, LANES)] = acc
            total = None
            stride_idx = iot * LANES
            for j in range(LANES):
                col = plsc.load_gather(pv, [stride_idx + j])
                total = col if total is None else total + col
            outb[pl.ds(t * B + g * LANES, LANES)] = 1.0 / (1.0 + jnp.exp(-total))
            return 0

        lax.fori_loop(0, GROUPS, group_body, 0)

    pltpu.sync_copy(e0_hbm.at[pl.ds(ebase, EDGES_PER_W)], idxa)
    pltpu.sync_copy(e1_hbm.at[pl.ds(ebase, EDGES_PER_W)], idxb)

    issue(0, 0, s0a, s0b)
    issue(1, 1, s1a, s1b)

    def pair_body(k, _):
        t = k * 2
        wait(t, 0, s0a, s0b)
        compute(t, 0)

        @pl.when(t + 2 < NBLOCKS)
        def _():
            issue(t + 2, 0, s0a, s0b)

        wait(t + 1, 1, s1a, s1b)
        compute(t + 1, 1)

        @pl.when(t + 3 < NBLOCKS)
        def _():
            issue(t + 3, 1, s1a, s1b)

        return 0

    lax.fori_loop(0, NBLOCKS // 2, pair_body, 0)
    wait(NBLOCKS - 1, 0, s0a, s0b)
    compute(NBLOCKS - 1, 0)

    pltpu.sync_copy(outb, out_hbm.at[pl.ds(ebase, EDGES_PER_W)])


def kernel(z, e):
    e0 = e[0]
    e1 = e[1]
    return _sc_kernel(z, e0, e1)
